# baseline (speedup 1.0000x reference)
import jax
import jax.numpy as jnp
from jax.experimental import pallas as pl
from jax.experimental.pallas import tpu as pltpu

_CHUNKS = (1024, 1024, 1024, 512, 256, 256)


def _mlp_body(x_hbm, m_ref, w_ref, w1_ref, b1_ref, w2_ref, b2_ref, w3_ref,
              b3_ref, o_hbm, xb, ob, insems, outsems):
    f32 = jnp.float32

    starts = []
    s = 0
    for c in _CHUNKS:
        starts.append(s)
        s += c

    in_copies = []
    for k, (st, sz) in enumerate(zip(starts, _CHUNKS)):
        c = pltpu.make_async_copy(x_hbm.at[pl.ds(st, sz), :],
                                  xb.at[pl.ds(st, sz), :], insems.at[k])
        c.start()
        in_copies.append(c)

    wm = w_ref[:] * m_ref[:].astype(f32)[:, None]
    b1 = b1_ref[:][None, :]
    b2 = b2_ref[:][None, :]
    b3 = b3_ref[:][None, :]

    out_copies = []
    for k, (st, sz) in enumerate(zip(starts, _CHUNKS)):
        in_copies[k].wait()
        h = jnp.dot(xb[pl.ds(st, sz), :], wm, preferred_element_type=f32)
        h = jnp.maximum(
            jnp.dot(h, w1_ref[:], preferred_element_type=f32) + b1, 0.0)
        h = jnp.maximum(
            jnp.dot(h, w2_ref[:], preferred_element_type=f32) + b2, 0.0)
        ob[pl.ds(st, sz), :] = (
            jnp.dot(h, w3_ref[:], preferred_element_type=f32) + b3)
        oc = pltpu.make_async_copy(ob.at[pl.ds(st, sz), :],
                                   o_hbm.at[pl.ds(st, sz), :], outsems.at[k])
        oc.start()
        out_copies.append(oc)

    for oc in out_copies:
        oc.wait()


def kernel(x, feature_mask, W, W1, b1, W2, b2, W3, b3):
    batch, feat = x.shape
    hidden = W.shape[1]
    classes = W3.shape[1]
    n = len(_CHUNKS)
    hbm = pl.BlockSpec(memory_space=pltpu.MemorySpace.HBM)
    vmem = pl.BlockSpec(memory_space=pltpu.MemorySpace.VMEM)
    return pl.pallas_call(
        _mlp_body,
        in_specs=[hbm, vmem, vmem, vmem, vmem, vmem, vmem, vmem, vmem],
        out_specs=hbm,
        out_shape=jax.ShapeDtypeStruct((batch, classes), x.dtype),
        scratch_shapes=[
            pltpu.VMEM((batch, feat), jnp.float32),
            pltpu.VMEM((batch, classes), jnp.float32),
            pltpu.SemaphoreType.DMA((n,)),
            pltpu.SemaphoreType.DMA((n,)),
        ],
    )(x, feature_mask, W, W1, b1, W2, b2, W3, b3)


# speedup vs baseline: 1.1140x; 1.1140x over previous
import jax
import jax.numpy as jnp
from jax.experimental import pallas as pl
from jax.experimental.pallas import tpu as pltpu

_BM = 1024


def _mlp_block(x_ref, m_ref, w_ref, w1_ref, b1_ref, w2_ref, b2_ref, w3_ref,
               b3_ref, o_ref):
    f32 = jnp.float32
    bf = jnp.bfloat16
    wm = (w_ref[:] * m_ref[:].astype(f32)[:, None]).astype(bf)
    h = jnp.dot(x_ref[:].astype(bf), wm, preferred_element_type=f32)
    h = jnp.maximum(
        jnp.dot(h.astype(bf), w1_ref[:].astype(bf),
                preferred_element_type=f32) + b1_ref[:][None, :], 0.0)
    h = jnp.maximum(
        jnp.dot(h.astype(bf), w2_ref[:].astype(bf),
                preferred_element_type=f32) + b2_ref[:][None, :], 0.0)
    o_ref[:] = (jnp.dot(h.astype(bf), w3_ref[:].astype(bf),
                        preferred_element_type=f32) + b3_ref[:][None, :])


def kernel(x, feature_mask, W, W1, b1, W2, b2, W3, b3):
    batch, feat = x.shape
    hidden = W.shape[1]
    classes = W3.shape[1]
    bm = min(_BM, batch)
    grid = (batch // bm,)
    full = lambda i: (0,)
    return pl.pallas_call(
        _mlp_block,
        grid=grid,
        compiler_params=pltpu.CompilerParams(
            dimension_semantics=("parallel",)),
        in_specs=[
            pl.BlockSpec((bm, feat), lambda i: (i, 0)),
            pl.BlockSpec((feat,), full),
            pl.BlockSpec((feat, hidden), lambda i: (0, 0)),
            pl.BlockSpec((hidden, hidden), lambda i: (0, 0)),
            pl.BlockSpec((hidden,), full),
            pl.BlockSpec((hidden, hidden), lambda i: (0, 0)),
            pl.BlockSpec((hidden,), full),
            pl.BlockSpec((hidden, classes), lambda i: (0, 0)),
            pl.BlockSpec((classes,), full),
        ],
        out_specs=pl.BlockSpec((bm, classes), lambda i: (i, 0)),
        out_shape=jax.ShapeDtypeStruct((batch, classes), x.dtype),
    )(x, feature_mask, W, W1, b1, W2, b2, W3, b3)
